# baseline (speedup 1.0000x reference)
import jax
import jax.numpy as jnp
from jax import lax
from jax.experimental import pallas as pl
from jax.experimental.pallas import tpu as pltpu
from jax.experimental.pallas import tpu_sc as plsc

VOCAB = 100000
EMBED = 64
BATCH = 4096
HIST = 50

NC = 2
NS = 16
LANES = 16
NJ = EMBED // LANES
NW = NC * NS
B_W = BATCH // NW
SPU = 2
IPU = SPU * HIST
U_W = B_W // SPU
NBUF = 4

def _gather_body(idx_hbm, table_hbm, out_hbm, raw_v, idx_v, rows_v, out_v,
                 *sems):
    wid = lax.axis_index("s") * NC + lax.axis_index("c")
    ubase = wid * U_W

    pltpu.sync_copy(idx_hbm.at[pl.ds(ubase, U_W)], raw_v)

    offs = [16 * m for m in range(IPU // 16)] + [IPU - 16]

    def urow(u, carry):
        for off in offs:
            v = raw_v[u, pl.ds(off, LANES)]
            r = (v & jnp.int32(~511)) + ((v & jnp.int32(255)) << 1) \
                + ((v >> 8) & jnp.int32(1))
            idx_v[u, pl.ds(off, LANES)] = r
        return carry

    lax.fori_loop(0, U_W, urow, 0)

    def fire(u, b):
        return pltpu.async_copy(table_hbm.at[idx_v.at[u]], rows_v.at[b], sems[b])

    for b in range(NBUF):
        fire(b, b)

    def group(gi, carry):
        for b in range(NBUF):
            u = gi * NBUF + b
            pltpu.make_async_copy(table_hbm.at[idx_v.at[u]], rows_v.at[b],
                                  sems[b]).wait()
            for p in range(SPU):
                base_t = p * HIST

                def tok(i, accs):
                    t = 2 * i
                    return tuple(
                        accs[k * NJ + j]
                        + rows_v[b, base_t + t + k, pl.ds(j * LANES, LANES)]
                        for k in range(2) for j in range(NJ)
                    )

                zero = jnp.zeros((LANES,), jnp.float32)
                accs = lax.fori_loop(0, HIST // 2, tok, (zero,) * (2 * NJ))
                s = SPU * u + p
                for j in range(NJ):
                    out_v[s, pl.ds(j * LANES, LANES)] = (
                        (accs[j] + accs[NJ + j]) * (1.0 / HIST))
            @pl.when(u + NBUF < U_W)
            def _():
                fire(u + NBUF, b)
        return carry

    lax.fori_loop(0, U_W // NBUF, group, 0)

    pltpu.sync_copy(out_v, out_hbm.at[pl.ds(wid * B_W, B_W)])


TCB = 512
TGRID = -(-VOCAB // TCB)
VPAD2 = TGRID * TCB


def _tc_transpose_body(x_ref, o_ref):
    x = x_ref[...]
    lo = x[:, :TCB // 2].T
    hi = x[:, TCB // 2:].T
    o_ref[...] = jnp.concatenate([lo, hi], axis=1).reshape(-1)


@jax.jit
def _emb_mean(idx, table):
    mesh = plsc.VectorSubcoreMesh(core_axis_name="c", subcore_axis_name="s")

    lin = pl.pallas_call(
        _tc_transpose_body,
        out_shape=jax.ShapeDtypeStruct((VPAD2 * EMBED,), jnp.float32),
        grid=(TGRID,),
        in_specs=[pl.BlockSpec((EMBED, TCB), lambda c: (0, c))],
        out_specs=pl.BlockSpec((TCB * EMBED,), lambda c: (c,)),
    )(table.T)

    table_lin = lin.reshape(VPAD2, EMBED)

    return pl.kernel(
        _gather_body,
        out_type=jax.ShapeDtypeStruct((BATCH, EMBED), jnp.float32),
        mesh=mesh,
        compiler_params=pltpu.CompilerParams(use_tc_tiling_on_sc=False,
                                             needs_layout_passes=False),
        scratch_types=[
            pltpu.VMEM((U_W, IPU), jnp.int32),
            pltpu.VMEM((U_W, IPU), jnp.int32),
            pltpu.VMEM((NBUF, IPU, EMBED), jnp.float32),
            pltpu.VMEM((B_W, EMBED), jnp.float32),
        ] + [pltpu.SemaphoreType.DMA] * NBUF,
    )(idx, table_lin)


def kernel(input_var, table):
    idx = input_var.astype(jnp.int32).reshape(BATCH // SPU, IPU)
    return _emb_mean(idx, table)


# speedup vs baseline: 1.5803x; 1.5803x over previous
import jax
import jax.numpy as jnp
from jax import lax
from jax.experimental import pallas as pl
from jax.experimental.pallas import tpu as pltpu
from jax.experimental.pallas import tpu_sc as plsc

VOCAB = 100000
EMBED = 64
BATCH = 4096
HIST = 50

NC = 2
NS = 16
LANES = 16
NJ = EMBED // LANES
NW = NC * NS
B_W = BATCH // NW
SPU = 2
IPU = SPU * HIST
U_W = B_W // SPU
NBUF = 4


def _gather_body(idx_hbm, table_hbm, out_hbm, idx_v, rows_v, out_v, *sems):
    wid = lax.axis_index("s") * NC + lax.axis_index("c")
    ubase = wid * U_W

    pltpu.sync_copy(idx_hbm.at[pl.ds(ubase, U_W)], idx_v)

    def fire(u, b):
        return pltpu.async_copy(table_hbm.at[idx_v.at[u]], rows_v.at[b], sems[b])

    for b in range(NBUF):
        fire(b, b)

    def group(gi, carry):
        for b in range(NBUF):
            u = gi * NBUF + b
            pltpu.make_async_copy(table_hbm.at[idx_v.at[u]], rows_v.at[b],
                                  sems[b]).wait()
            for p in range(SPU):
                base_t = p * HIST

                def tok(i, accs):
                    t = 2 * i
                    return tuple(
                        accs[k * NJ + j]
                        + rows_v[b, base_t + t + k, pl.ds(j * LANES, LANES)]
                        for k in range(2) for j in range(NJ)
                    )

                zero = jnp.zeros((LANES,), jnp.float32)
                accs = lax.fori_loop(0, HIST // 2, tok, (zero,) * (2 * NJ))
                s = SPU * u + p
                for j in range(NJ):
                    out_v[s, pl.ds(j * LANES, LANES)] = (
                        (accs[j] + accs[NJ + j]) * (1.0 / HIST))
            @pl.when(u + NBUF < U_W)
            def _():
                fire(u + NBUF, b)
        return carry

    lax.fori_loop(0, U_W // NBUF, group, 0)

    pltpu.sync_copy(out_v, out_hbm.at[pl.ds(wid * B_W, B_W)])


@jax.jit
def _emb_mean(idx, table):
    mesh = plsc.VectorSubcoreMesh(core_axis_name="c", subcore_axis_name="s")
    return pl.kernel(
        _gather_body,
        out_type=jax.ShapeDtypeStruct((BATCH, EMBED), jnp.float32),
        mesh=mesh,
        compiler_params=pltpu.CompilerParams(use_tc_tiling_on_sc=False),
        scratch_types=[
            pltpu.VMEM((U_W, IPU), jnp.int32),
            pltpu.VMEM((NBUF, IPU, EMBED), jnp.float32),
            pltpu.VMEM((B_W, EMBED), jnp.float32),
        ] + [pltpu.SemaphoreType.DMA] * NBUF,
    )(idx, table)


def kernel(input_var, table):
    idx = input_var.astype(jnp.int32).reshape(BATCH // SPU, IPU)
    return _emb_mean(idx, table)
